# Initial kernel scaffold; baseline (speedup 1.0000x reference)
import jax
import jax.numpy as jnp
from jax.experimental import pallas as pl


def kernel(z_e, codebook):
    raise NotImplementedError("write your pallas kernel here")


# speedup vs baseline: 3.4684x; 3.4684x over previous
import jax
import jax.numpy as jnp
from jax.experimental import pallas as pl


def _vq_block(z_ref, cb_ref, zq_ref, idx_ref):
    z = z_ref[...]
    cb = cb_ref[...]
    c2 = jnp.sum(cb * cb, axis=1, keepdims=True)
    a = jnp.concatenate([-2.0 * z, jnp.ones((z.shape[0], 1), jnp.float32)], axis=1)
    caug = jnp.concatenate([cb, c2], axis=1)
    d = jax.lax.dot_general(
        a, caug, (((1,), (1,)), ((), ())),
        precision=jax.lax.Precision.HIGHEST,
        preferred_element_type=jnp.float32)
    m = jnp.min(d, axis=1, keepdims=True)
    k = d.shape[1]
    iota = jax.lax.broadcasted_iota(jnp.int32, d.shape, 1)
    idx = jnp.min(jnp.where(d == m, iota, k), axis=1)
    idx_ref[...] = idx[:, None]
    onehot = (iota == idx[:, None]).astype(jnp.float32)
    zq = jax.lax.dot_general(
        onehot, cb, (((1,), (0,)), ((), ())),
        precision=jax.lax.Precision.HIGHEST,
        preferred_element_type=jnp.float32)
    zq_ref[...] = z + (zq - z)


def kernel(z_e, codebook):
    b, t, d = z_e.shape
    k = codebook.shape[0]
    n = b * t
    blocks = 7
    tb = n // blocks
    zf = z_e.reshape(n, d)
    zq, idx = pl.pallas_call(
        _vq_block,
        grid=(blocks,),
        in_specs=[
            pl.BlockSpec((tb, d), lambda i: (i, 0)),
            pl.BlockSpec((k, d), lambda i: (0, 0)),
        ],
        out_specs=[
            pl.BlockSpec((tb, d), lambda i: (i, 0)),
            pl.BlockSpec((tb, 1), lambda i: (i, 0)),
        ],
        out_shape=[
            jax.ShapeDtypeStruct((n, d), jnp.float32),
            jax.ShapeDtypeStruct((n, 1), jnp.int32),
        ],
    )(zf, codebook)
    return zq.reshape(b, t, d), idx.reshape(b, t)


# speedup vs baseline: 4.2167x; 1.2158x over previous
import jax
import jax.numpy as jnp
from jax.experimental import pallas as pl
from jax.experimental.pallas import tpu as pltpu


def _vq_block(z_ref, cb_ref, zq_ref, idx_ref, caug_ref):
    @pl.when(pl.program_id(0) == 0)
    def _init():
        cb = cb_ref[...]
        c2 = jnp.sum(cb * cb, axis=1, keepdims=True)
        caug_ref[...] = jnp.concatenate([cb, c2], axis=1)

    z = z_ref[...]
    caug = caug_ref[...]
    a = jnp.concatenate([-2.0 * z, jnp.ones((z.shape[0], 1), jnp.float32)], axis=1)
    d = jax.lax.dot_general(
        a, caug, (((1,), (1,)), ((), ())),
        precision=jax.lax.Precision.HIGHEST,
        preferred_element_type=jnp.float32)
    idx = jnp.argmin(d, axis=1)
    idx_ref[...] = idx[:, None]
    iota = jax.lax.broadcasted_iota(jnp.int32, d.shape, 1)
    onehot = (iota == idx[:, None]).astype(jnp.float32)
    zq = jax.lax.dot_general(
        onehot, cb_ref[...], (((1,), (0,)), ((), ())),
        precision=jax.lax.Precision.HIGHEST,
        preferred_element_type=jnp.float32)
    zq_ref[...] = z + (zq - z)


def kernel(z_e, codebook):
    b, t, d = z_e.shape
    k = codebook.shape[0]
    n = b * t
    blocks = 14
    tb = n // blocks
    zf = z_e.reshape(n, d)
    zq, idx = pl.pallas_call(
        _vq_block,
        grid=(blocks,),
        in_specs=[
            pl.BlockSpec((tb, d), lambda i: (i, 0)),
            pl.BlockSpec((k, d), lambda i: (0, 0)),
        ],
        out_specs=[
            pl.BlockSpec((tb, d), lambda i: (i, 0)),
            pl.BlockSpec((tb, 1), lambda i: (i, 0)),
        ],
        out_shape=[
            jax.ShapeDtypeStruct((n, d), jnp.float32),
            jax.ShapeDtypeStruct((n, 1), jnp.int32),
        ],
        scratch_shapes=[pltpu.VMEM((k, d + 1), jnp.float32)],
    )(zf, codebook)
    return zq.reshape(b, t, d), idx.reshape(b, t)
